# baseline (speedup 1.0000x reference)
import functools
import math

import jax
import jax.numpy as jnp
from jax import lax
from jax.experimental import pallas as pl
from jax.experimental.pallas import tpu as pltpu
from jax.experimental.pallas import tpu_sc as plsc

VOCAB = 32128
D = 1024
N_TOK = 4 * 4096
NC, NS = 2, 16
NW = NC * NS
B_PER_W = N_TOK // NW
CHUNK = 32
N_CHUNK = B_PER_W // CHUNK
SCALE = math.sqrt(D)
LANES = 16


NBUF = 3


def _body(table_hbm, ids_hbm, out_hbm, idx_v, rows0, rows1, rows2, gsem0,
          gsem1, gsem2, wsem0, wsem1, wsem2):
    wid = lax.axis_index("s") * NC + lax.axis_index("c")
    rows = (rows0, rows1, rows2)
    gsem = (gsem0, gsem1, gsem2)
    wsem = (wsem0, wsem1, wsem2)

    w_per_row = ids_hbm.shape[1] // B_PER_W
    row = wid // w_per_row
    col = (wid % w_per_row) * B_PER_W
    pltpu.sync_copy(ids_hbm.at[row, pl.ds(col, B_PER_W)], idx_v)

    def scale_chunk(buf):
        @plsc.parallel_loop(0, CHUNK * D // LANES, unroll=8)
        def _(i):
            r = i // (D // LANES)
            c = i % (D // LANES)
            sl = pl.ds(c * LANES, LANES)
            buf[r, sl] = buf[r, sl] * SCALE

    def start_gather(g):
        b = g % NBUF
        return pltpu.async_copy(
            table_hbm.at[idx_v.at[pl.ds(g * CHUNK, CHUNK)]], rows[b], gsem[b])

    gathers = [None] * N_CHUNK
    writes = [None] * N_CHUNK
    gathers[0] = start_gather(0)
    gathers[1] = start_gather(1)
    for g in range(N_CHUNK):
        b = g % NBUF
        if g + 2 < N_CHUNK:
            if g >= 1 and writes[g - 1] is not None:
                writes[g - 1].wait()
            gathers[g + 2] = start_gather(g + 2)
        gathers[g].wait()
        scale_chunk(rows[b])
        writes[g] = pltpu.async_copy(
            rows[b], out_hbm.at[pl.ds(wid * B_PER_W + g * CHUNK, CHUNK)],
            wsem[b])
    writes[N_CHUNK - 3].wait()
    writes[N_CHUNK - 2].wait()
    writes[N_CHUNK - 1].wait()


@functools.partial(jax.jit, static_argnames=())
def kernel(input_ids, token_embedding):
    ids = input_ids
    if ids.dtype != jnp.int32:
        ids = ids.astype(jnp.int32)
    run = pl.kernel(
        _body,
        out_type=jax.ShapeDtypeStruct((N_TOK, D), jnp.float32),
        mesh=plsc.VectorSubcoreMesh(core_axis_name="c", subcore_axis_name="s"),
        scratch_types=[
            pltpu.VMEM((B_PER_W,), jnp.int32),
            pltpu.VMEM((CHUNK, D), jnp.float32),
            pltpu.VMEM((CHUNK, D), jnp.float32),
            pltpu.VMEM((CHUNK, D), jnp.float32),
            pltpu.SemaphoreType.DMA,
            pltpu.SemaphoreType.DMA,
            pltpu.SemaphoreType.DMA,
            pltpu.SemaphoreType.DMA,
            pltpu.SemaphoreType.DMA,
            pltpu.SemaphoreType.DMA,
        ],
    )
    out = run(token_embedding, ids)
    return out.reshape(input_ids.shape[0], input_ids.shape[1], D)


# speedup vs baseline: 1.0194x; 1.0194x over previous
import functools
import math

import jax
import jax.numpy as jnp
from jax import lax
from jax.experimental import pallas as pl
from jax.experimental.pallas import tpu as pltpu
from jax.experimental.pallas import tpu_sc as plsc

VOCAB = 32128
D = 1024
N_TOK = 4 * 4096
NC, NS = 2, 16
NW = NC * NS
B_PER_W = N_TOK // NW
CHUNK = 32
N_CHUNK = B_PER_W // CHUNK
SCALE = math.sqrt(D)
LANES = 16


def _body(table_hbm, ids_hbm, out_hbm, idx_v, rows0, rows1, gsem0, gsem1,
          wsem0, wsem1):
    wid = lax.axis_index("s") * NC + lax.axis_index("c")
    rows = (rows0, rows1)
    gsem = (gsem0, gsem1)
    wsem = (wsem0, wsem1)

    w_per_row = ids_hbm.shape[1] // B_PER_W
    row = wid // w_per_row
    col = (wid % w_per_row) * B_PER_W
    pltpu.sync_copy(ids_hbm.at[row, pl.ds(col, B_PER_W)], idx_v)

    def idx_slice(g):
        return idx_v.at[pl.ds(pl.multiple_of(g * CHUNK, CHUNK), CHUNK)]

    def start_gather(g, b):
        return pltpu.async_copy(table_hbm.at[idx_slice(g)], rows[b], gsem[b])

    def scale_chunk(b):
        buf = rows[b]

        @plsc.parallel_loop(0, CHUNK * D // LANES, unroll=8)
        def _(i):
            r = i // (D // LANES)
            c = i % (D // LANES)
            sl = pl.ds(c * LANES, LANES)
            buf[r, sl] = buf[r, sl] * SCALE

    def out_slice(g):
        off = pl.multiple_of(wid * B_PER_W + g * CHUNK, CHUNK)
        return out_hbm.at[pl.ds(off, CHUNK)]

    def wait_write(g, b):
        pltpu.make_async_copy(rows[b], out_slice(g), wsem[b]).wait()

    def chunk_body(g, b, first=False, last=False):
        if not first:
            wait_write(g - 1, 1 - b)
        if not last:
            start_gather(g + 1, 1 - b)
        pltpu.make_async_copy(
            table_hbm.at[idx_slice(g)], rows[b], gsem[b]).wait()
        scale_chunk(b)
        pltpu.async_copy(rows[b], out_slice(g), wsem[b])

    start_gather(0, 0)
    chunk_body(0, 0, first=True)

    @pl.loop(1, N_CHUNK - 1, step=2)
    def _(g):
        chunk_body(g, 1)
        chunk_body(g + 1, 0)

    chunk_body(N_CHUNK - 1, 1, last=True)
    wait_write(N_CHUNK - 1, 1)


@functools.partial(jax.jit, static_argnames=())
def kernel(input_ids, token_embedding):
    ids = input_ids
    if ids.dtype != jnp.int32:
        ids = ids.astype(jnp.int32)
    run = pl.kernel(
        _body,
        out_type=jax.ShapeDtypeStruct((N_TOK, D), jnp.float32),
        mesh=plsc.VectorSubcoreMesh(core_axis_name="c", subcore_axis_name="s"),
        scratch_types=[
            pltpu.VMEM((B_PER_W,), jnp.int32),
            pltpu.VMEM((CHUNK, D), jnp.float32),
            pltpu.VMEM((CHUNK, D), jnp.float32),
            pltpu.SemaphoreType.DMA,
            pltpu.SemaphoreType.DMA,
            pltpu.SemaphoreType.DMA,
            pltpu.SemaphoreType.DMA,
        ],
    )
    out = run(token_embedding, ids)
    return out.reshape(input_ids.shape[0], input_ids.shape[1], D)


# speedup vs baseline: 1.0274x; 1.0079x over previous
import functools
import math

import jax
import jax.numpy as jnp
from jax import lax
from jax.experimental import pallas as pl
from jax.experimental.pallas import tpu as pltpu
from jax.experimental.pallas import tpu_sc as plsc

VOCAB = 32128
D = 1024
N_TOK = 4 * 4096
NC, NS = 2, 16
NW = NC * NS
B_PER_W = N_TOK // NW
CHUNK = 32
N_CHUNK = B_PER_W // CHUNK
SCALE = math.sqrt(D)
LANES = 16


def _body(table_hbm, ids_hbm, out_hbm, idx_v, rows0, rows1, gsem0, gsem1,
          wsem0, wsem1):
    wid = lax.axis_index("s") * NC + lax.axis_index("c")
    rows = (rows0, rows1)
    gsem = (gsem0, gsem1)
    wsem = (wsem0, wsem1)

    w_per_row = ids_hbm.shape[1] // B_PER_W
    row = wid // w_per_row
    col = (wid % w_per_row) * B_PER_W
    pltpu.sync_copy(ids_hbm.at[row, pl.ds(col, 128)],
                    idx_v.at[pl.ds(0, 128)])

    def idx_slice(g):
        return idx_v.at[pl.ds(pl.multiple_of(g * CHUNK, CHUNK), CHUNK)]

    def start_gather(g, b):
        return pltpu.async_copy(table_hbm.at[idx_slice(g)], rows[b], gsem[b])

    def scale_chunk(b):
        buf = rows[b]

        @plsc.parallel_loop(0, CHUNK * D // LANES, unroll=8)
        def _(i):
            r = i // (D // LANES)
            c = i % (D // LANES)
            sl = pl.ds(c * LANES, LANES)
            buf[r, sl] = buf[r, sl] * SCALE

    def out_slice(g):
        off = pl.multiple_of(wid * B_PER_W + g * CHUNK, CHUNK)
        return out_hbm.at[pl.ds(off, CHUNK)]

    def wait_write(g, b):
        pltpu.make_async_copy(rows[b], out_slice(g), wsem[b]).wait()

    def chunk_body(g, b, first=False, last=False):
        if not first:
            wait_write(g - 1, 1 - b)
        if not last:
            start_gather(g + 1, 1 - b)
        pltpu.make_async_copy(
            table_hbm.at[idx_slice(g)], rows[b], gsem[b]).wait()
        scale_chunk(b)
        pltpu.async_copy(rows[b], out_slice(g), wsem[b])

    start_gather(0, 0)
    pltpu.sync_copy(ids_hbm.at[row, pl.ds(col + 128, B_PER_W - 128)],
                    idx_v.at[pl.ds(128, B_PER_W - 128)])
    chunk_body(0, 0, first=True)

    @pl.loop(1, N_CHUNK - 1, step=2)
    def _(g):
        chunk_body(g, 1)
        chunk_body(g + 1, 0)

    chunk_body(N_CHUNK - 1, 1, last=True)
    wait_write(N_CHUNK - 1, 1)


@functools.partial(jax.jit, static_argnames=())
def kernel(input_ids, token_embedding):
    ids = input_ids
    if ids.dtype != jnp.int32:
        ids = ids.astype(jnp.int32)
    run = pl.kernel(
        _body,
        out_type=jax.ShapeDtypeStruct((N_TOK, D), jnp.float32),
        mesh=plsc.VectorSubcoreMesh(core_axis_name="c", subcore_axis_name="s"),
        scratch_types=[
            pltpu.VMEM((B_PER_W,), jnp.int32),
            pltpu.VMEM((CHUNK, D), jnp.float32),
            pltpu.VMEM((CHUNK, D), jnp.float32),
            pltpu.SemaphoreType.DMA,
            pltpu.SemaphoreType.DMA,
            pltpu.SemaphoreType.DMA,
            pltpu.SemaphoreType.DMA,
        ],
    )
    out = run(token_embedding, ids)
    return out.reshape(input_ids.shape[0], input_ids.shape[1], D)


# speedup vs baseline: 1.0435x; 1.0157x over previous
import functools
import math

import jax
import jax.numpy as jnp
from jax import lax
from jax.experimental import pallas as pl
from jax.experimental.pallas import tpu as pltpu
from jax.experimental.pallas import tpu_sc as plsc

VOCAB = 32128
D = 1024
N_TOK = 4 * 4096
NC, NS = 2, 16
NW = NC * NS
B_PER_W = N_TOK // NW
CHUNK = 32
N_CHUNK = B_PER_W // CHUNK
SCALE = math.sqrt(D)
LANES = 16


NBUF = 3


def _body(table_hbm, ids_hbm, out_hbm, idx_v, rows0, rows1, rows2, gsem0,
          gsem1, gsem2, wsem0, wsem1, wsem2):
    wid = lax.axis_index("s") * NC + lax.axis_index("c")
    rows = (rows0, rows1, rows2)
    gsem = (gsem0, gsem1, gsem2)
    wsem = (wsem0, wsem1, wsem2)

    w_per_row = ids_hbm.shape[1] // B_PER_W
    row = wid // w_per_row
    col = (wid % w_per_row) * B_PER_W
    pltpu.sync_copy(ids_hbm.at[row, pl.ds(col, 128)],
                    idx_v.at[pl.ds(0, 128)])

    def idx_slice(g):
        return idx_v.at[pl.ds(pl.multiple_of(g * CHUNK, CHUNK), CHUNK)]

    def start_gather(g, b):
        return pltpu.async_copy(table_hbm.at[idx_slice(g)], rows[b], gsem[b])

    def scale_chunk(b):
        buf = rows[b]

        @plsc.parallel_loop(0, CHUNK * D // LANES, unroll=8)
        def _(i):
            r = i // (D // LANES)
            c = i % (D // LANES)
            sl = pl.ds(c * LANES, LANES)
            buf[r, sl] = buf[r, sl] * SCALE

    def out_slice(g):
        off = pl.multiple_of(wid * B_PER_W + g * CHUNK, CHUNK)
        return out_hbm.at[pl.ds(off, CHUNK)]

    def wait_write(g, b):
        pltpu.make_async_copy(rows[b], out_slice(g), wsem[b]).wait()

    def chunk_body(g, b, drain=True, prefetch=True):
        if prefetch:
            if drain:
                wait_write(g - 1, (b + 2) % NBUF)
            start_gather(g + 2, (b + 2) % NBUF)
        pltpu.make_async_copy(
            table_hbm.at[idx_slice(g)], rows[b], gsem[b]).wait()
        scale_chunk(b)
        pltpu.async_copy(rows[b], out_slice(g), wsem[b])

    start_gather(0, 0)
    pltpu.sync_copy(ids_hbm.at[row, pl.ds(col + 128, B_PER_W - 128)],
                    idx_v.at[pl.ds(128, B_PER_W - 128)])
    start_gather(1, 1)
    chunk_body(0, 0, drain=False)
    chunk_body(1, 1)

    @pl.loop(2, N_CHUNK - 2, step=NBUF)
    def _(g):
        chunk_body(g, 2)
        chunk_body(g + 1, 0)
        chunk_body(g + 2, 1)

    chunk_body(N_CHUNK - 2, 2, prefetch=False)
    chunk_body(N_CHUNK - 1, 0, prefetch=False)
    wait_write(N_CHUNK - 3, 1)
    wait_write(N_CHUNK - 2, 2)
    wait_write(N_CHUNK - 1, 0)


@functools.partial(jax.jit, static_argnames=())
def kernel(input_ids, token_embedding):
    ids = input_ids
    if ids.dtype != jnp.int32:
        ids = ids.astype(jnp.int32)
    run = pl.kernel(
        _body,
        out_type=jax.ShapeDtypeStruct((N_TOK, D), jnp.float32),
        mesh=plsc.VectorSubcoreMesh(core_axis_name="c", subcore_axis_name="s"),
        scratch_types=[
            pltpu.VMEM((B_PER_W,), jnp.int32),
            pltpu.VMEM((CHUNK, D), jnp.float32),
            pltpu.VMEM((CHUNK, D), jnp.float32),
            pltpu.VMEM((CHUNK, D), jnp.float32),
            pltpu.SemaphoreType.DMA,
            pltpu.SemaphoreType.DMA,
            pltpu.SemaphoreType.DMA,
            pltpu.SemaphoreType.DMA,
            pltpu.SemaphoreType.DMA,
            pltpu.SemaphoreType.DMA,
        ],
    )
    out = run(token_embedding, ids)
    return out.reshape(input_ids.shape[0], input_ids.shape[1], D)
